# Initial kernel scaffold; baseline (speedup 1.0000x reference)
import jax
import jax.numpy as jnp
from jax.experimental import pallas as pl


def kernel(indices, table):
    raise NotImplementedError("write your pallas kernel here")


# speedup vs baseline: 4.1092x; 4.1092x over previous
import functools

import jax
import jax.numpy as jnp
from jax import lax
from jax.experimental import pallas as pl
from jax.experimental.pallas import tpu as pltpu
from jax.experimental.pallas import tpu_sc as plsc

NUM_EMB = 100000
DIM = 64
BATCH = 4096
HIST = 200

_info = plsc.get_sparse_core_info()
NC, NS = _info.num_cores, _info.num_subcores
NW = NC * NS
TOTAL = BATCH * HIST
PER_W = TOTAL // NW
CHUNK = 128
NCHUNK = PER_W // CHUNK
NPAIR = NCHUNK // 2


def _sc_gather(idx_hbm, table_hbm, out_hbm, idx_v, rows0, rows1, sem0, sem1):
    wid = lax.axis_index("s") * NC + lax.axis_index("c")
    base = wid * PER_W

    pltpu.sync_copy(idx_hbm.at[wid], idx_v)

    def gather(g, rows, sem):
        pltpu.make_async_copy(table_hbm.at[idx_v.at[g]], rows, sem).start()

    def put(g, rows):
        pltpu.sync_copy(rows, out_hbm.at[pl.ds(base + g * CHUNK, CHUNK)])

    gather(0, rows0, sem0)
    gather(1, rows1, sem1)

    def step(t, carry):
        for b, (rows, sem) in enumerate(((rows0, sem0), (rows1, sem1))):
            g = 2 * t + b
            pltpu.make_async_copy(table_hbm.at[idx_v.at[g]], rows, sem).wait()
            put(g, rows)
            gather(g + 2, rows, sem)
        return carry

    lax.fori_loop(0, NPAIR - 1, step, 0, unroll=False)

    for b, (rows, sem) in enumerate(((rows0, sem0), (rows1, sem1))):
        g = 2 * (NPAIR - 1) + b
        pltpu.make_async_copy(table_hbm.at[idx_v.at[g]], rows, sem).wait()
        put(g, rows)


@jax.jit
def kernel(indices, table):
    idx = indices.reshape(NW, NCHUNK, CHUNK).astype(jnp.int32)
    mesh = plsc.VectorSubcoreMesh(core_axis_name="c", subcore_axis_name="s")
    out = pl.kernel(
        _sc_gather,
        mesh=mesh,
        out_type=jax.ShapeDtypeStruct((TOTAL, DIM), jnp.float32),
        scratch_types=[
            pltpu.VMEM((NCHUNK, CHUNK), jnp.int32),
            pltpu.VMEM((CHUNK, DIM), jnp.float32),
            pltpu.VMEM((CHUNK, DIM), jnp.float32),
            pltpu.SemaphoreType.DMA,
            pltpu.SemaphoreType.DMA,
        ],
        compiler_params=pltpu.CompilerParams(use_tc_tiling_on_sc=False),
    )(idx, table)
    return out.reshape(BATCH, HIST, DIM)


# speedup vs baseline: 4.2655x; 1.0380x over previous
import functools

import jax
import jax.numpy as jnp
from jax import lax
from jax.experimental import pallas as pl
from jax.experimental.pallas import tpu as pltpu
from jax.experimental.pallas import tpu_sc as plsc

NUM_EMB = 100000
DIM = 64
BATCH = 4096
HIST = 200

_info = plsc.get_sparse_core_info()
NC, NS = _info.num_cores, _info.num_subcores
NW = NC * NS
TOTAL = BATCH * HIST
PER_W = TOTAL // NW
CHUNK = 128
NCHUNK = PER_W // CHUNK
K = 4
SLOT = K * CHUNK
NFILL = NCHUNK // K


def _sc_gather(idx_hbm, table_hbm, out_hbm,
               idx_v, rows0, rows1, gs0, gs1, os0, os1):
    wid = lax.axis_index("s") * NC + lax.axis_index("c")
    base = wid * PER_W

    pltpu.sync_copy(idx_hbm.at[wid], idx_v)

    def fill(s, rows, gsem):
        for j in range(K):
            g = s * K + j
            pltpu.make_async_copy(
                table_hbm.at[idx_v.at[g]],
                rows.at[pl.ds(j * CHUNK, CHUNK)], gsem).start()

    def drain_fill(rows, gsem):
        pltpu.make_async_copy(table_hbm.at[pl.ds(0, SLOT)], rows, gsem).wait()

    def out_slice(s):
        return out_hbm.at[pl.ds(base + s * SLOT, SLOT)]

    def put(s, rows, osem):
        pltpu.make_async_copy(rows, out_slice(s), osem).start()

    def wait_put(s, rows, osem):
        pltpu.make_async_copy(rows, out_slice(s), osem).wait()

    fill(0, rows0, gs0)
    fill(1, rows1, gs1)

    def step(t, carry):
        for b, (rows, gsem, osem) in enumerate(
                ((rows0, gs0, os0), (rows1, gs1, os1))):
            s = 2 * t + b
            drain_fill(rows, gsem)
            put(s, rows, osem)

            @pl.when(s + 2 < NFILL)
            def _():
                wait_put(s, rows, osem)
                fill(s + 2, rows, gsem)
        return carry

    lax.fori_loop(0, NFILL // 2, step, 0, unroll=False)

    wait_put(NFILL - 2, rows0, os0)
    wait_put(NFILL - 1, rows1, os1)


@jax.jit
def kernel(indices, table):
    idx = indices.reshape(NW, NCHUNK, CHUNK).astype(jnp.int32)
    mesh = plsc.VectorSubcoreMesh(core_axis_name="c", subcore_axis_name="s")
    out = pl.kernel(
        _sc_gather,
        mesh=mesh,
        out_type=jax.ShapeDtypeStruct((TOTAL, DIM), jnp.float32),
        scratch_types=[
            pltpu.VMEM((NCHUNK, CHUNK), jnp.int32),
            pltpu.VMEM((SLOT, DIM), jnp.float32),
            pltpu.VMEM((SLOT, DIM), jnp.float32),
            pltpu.SemaphoreType.DMA,
            pltpu.SemaphoreType.DMA,
            pltpu.SemaphoreType.DMA,
            pltpu.SemaphoreType.DMA,
        ],
        compiler_params=pltpu.CompilerParams(use_tc_tiling_on_sc=False),
    )(idx, table)
    return out.reshape(BATCH, HIST, DIM)
